# baseline (speedup 1.0000x reference)
import functools

import jax
import jax.numpy as jnp
from jax import lax
from jax.experimental import pallas as pl
from jax.experimental.pallas import tpu as pltpu
from jax.experimental.pallas import tpu_sc as plsc

_ROWS = 8192
_DIM = 1024

_SC_ROWS = 2560
_TC_ROWS = _ROWS - _SC_ROWS
_TC_BLOCK = 512

_NW = 32
_RPW = _SC_ROWS // _NW
_SC_CHUNK = 40
_NCH = _RPW // _SC_CHUNK


@functools.partial(
    pl.kernel,
    mesh=plsc.VectorSubcoreMesh(core_axis_name="c", subcore_axis_name="s"),
    out_type=jax.ShapeDtypeStruct((_SC_ROWS, _DIM), jnp.float32),
    scratch_types=[
        pltpu.VMEM((2, _SC_CHUNK, _DIM), jnp.float32),
        pltpu.SemaphoreType.DMA((2,)),
        pltpu.SemaphoreType.DMA((2,)),
    ],
)
def _sc_copy_tail(emb_hbm, out_hbm, buf, in_sems, out_sems):
    wid = lax.axis_index("s") * 2 + lax.axis_index("c")
    src_base = _TC_ROWS + wid * _RPW
    dst_base = wid * _RPW

    def in_copy(g, slot):
        return pltpu.make_async_copy(
            emb_hbm.at[pl.ds(src_base + g * _SC_CHUNK, _SC_CHUNK), :],
            buf.at[slot],
            in_sems.at[slot],
        )

    def out_copy(g, slot):
        return pltpu.make_async_copy(
            buf.at[slot],
            out_hbm.at[pl.ds(dst_base + g * _SC_CHUNK, _SC_CHUNK), :],
            out_sems.at[slot],
        )

    in_copy(0, 0).start()
    for g in range(_NCH):
        slot = g % 2
        in_copy(g, slot).wait()
        out_copy(g, slot).start()
        if g + 1 < _NCH:
            nslot = (g + 1) % 2
            if g >= 1:
                out_copy(g - 1, nslot).wait()
            in_copy(g + 1, nslot).start()
    if _NCH >= 2:
        out_copy(_NCH - 2, (_NCH - 2) % 2).wait()
    out_copy(_NCH - 1, (_NCH - 1) % 2).wait()


def _tc_body(in_ref, out_ref):
    out_ref[...] = in_ref[...]


def _tc_copy_head(emb):
    return pl.pallas_call(
        _tc_body,
        grid=(_TC_ROWS // _TC_BLOCK,),
        in_specs=[pl.BlockSpec((_TC_BLOCK, _DIM), lambda i: (i, 0))],
        out_specs=pl.BlockSpec((_TC_BLOCK, _DIM), lambda i: (i, 0)),
        out_shape=jax.ShapeDtypeStruct((_ROWS, _DIM), jnp.float32),
    )(emb)


def kernel(x, emb):
    del x
    head = _tc_copy_head(emb)
    tail = _sc_copy_tail(emb)
    return lax.dynamic_update_slice(head, tail, (_TC_ROWS, 0))


# speedup vs baseline: 1.1084x; 1.1084x over previous
import functools

import jax
import jax.numpy as jnp
from jax import lax
from jax.experimental import pallas as pl
from jax.experimental.pallas import tpu as pltpu
from jax.experimental.pallas import tpu_sc as plsc

_ROWS = 8192
_DIM = 1024
_NW = 32
_RPW = _ROWS // _NW
_SC_CHUNK = 32
_NCH = _RPW // _SC_CHUNK
_NBUF = 3


@functools.partial(
    pl.kernel,
    mesh=plsc.VectorSubcoreMesh(core_axis_name="c", subcore_axis_name="s"),
    out_type=jax.ShapeDtypeStruct((_ROWS, _DIM), jnp.float32),
    scratch_types=[
        pltpu.VMEM((_NBUF, _SC_CHUNK, _DIM), jnp.float32),
        pltpu.SemaphoreType.DMA((_NBUF,)),
        pltpu.SemaphoreType.DMA((_NBUF,)),
    ],
)
def _sc_copy(emb_hbm, out_hbm, buf, in_sems, out_sems):
    wid = lax.axis_index("s") * 2 + lax.axis_index("c")
    base = wid * _RPW

    def in_copy(g):
        slot = g % _NBUF
        return pltpu.make_async_copy(
            emb_hbm.at[pl.ds(base + g * _SC_CHUNK, _SC_CHUNK), :],
            buf.at[slot],
            in_sems.at[slot],
        )

    def out_copy(g):
        slot = g % _NBUF
        return pltpu.make_async_copy(
            buf.at[slot],
            out_hbm.at[pl.ds(base + g * _SC_CHUNK, _SC_CHUNK), :],
            out_sems.at[slot],
        )

    in_copy(0).start()
    for g in range(_NCH):
        in_copy(g).wait()
        out_copy(g).start()
        if g + 1 < _NCH:
            if g + 1 >= _NBUF:
                out_copy(g + 1 - _NBUF).wait()
            in_copy(g + 1).start()
    for g in range(max(0, _NCH - _NBUF), _NCH):
        out_copy(g).wait()


def kernel(x, emb):
    del x
    return _sc_copy(emb)


# speedup vs baseline: 1.1465x; 1.0344x over previous
import functools

import jax
import jax.numpy as jnp
from jax import lax
from jax.experimental import pallas as pl
from jax.experimental.pallas import tpu as pltpu
from jax.experimental.pallas import tpu_sc as plsc

_ROWS = 8192
_DIM = 1024
_NW = 32
_RPW = _ROWS // _NW
_SC_CHUNK = 64
_NCH = _RPW // _SC_CHUNK
_NBUF = 2


@functools.partial(
    pl.kernel,
    mesh=plsc.VectorSubcoreMesh(core_axis_name="c", subcore_axis_name="s"),
    out_type=jax.ShapeDtypeStruct((_ROWS, _DIM), jnp.float32),
    scratch_types=[
        pltpu.VMEM_SHARED((16, _NBUF, _SC_CHUNK, _DIM), jnp.float32),
        pltpu.SemaphoreType.DMA((_NBUF,)),
        pltpu.SemaphoreType.DMA((_NBUF,)),
    ],
)
def _sc_copy(emb_hbm, out_hbm, sbuf, in_sems, out_sems):
    sid = lax.axis_index("s")
    wid = sid * 2 + lax.axis_index("c")
    base = wid * _RPW
    buf = sbuf.at[sid]

    def in_copy(g):
        slot = g % _NBUF
        return pltpu.make_async_copy(
            emb_hbm.at[pl.ds(base + g * _SC_CHUNK, _SC_CHUNK), :],
            buf.at[slot],
            in_sems.at[slot],
        )

    def out_copy(g):
        slot = g % _NBUF
        return pltpu.make_async_copy(
            buf.at[slot],
            out_hbm.at[pl.ds(base + g * _SC_CHUNK, _SC_CHUNK), :],
            out_sems.at[slot],
        )

    in_copy(0).start()
    for g in range(_NCH):
        in_copy(g).wait()
        out_copy(g).start()
        if g + 1 < _NCH:
            if g + 1 >= _NBUF:
                out_copy(g + 1 - _NBUF).wait()
            in_copy(g + 1).start()
    for g in range(max(0, _NCH - _NBUF), _NCH):
        out_copy(g).wait()


def kernel(x, emb):
    del x
    return _sc_copy(emb)


# speedup vs baseline: 1.1665x; 1.0174x over previous
import functools

import jax
import jax.numpy as jnp
from jax import lax
from jax.experimental import pallas as pl
from jax.experimental.pallas import tpu as pltpu
from jax.experimental.pallas import tpu_sc as plsc

_ROWS = 8192
_DIM = 1024
_NW = 32
_RPW = _ROWS // _NW
_SC_CHUNK = 32
_NCH = _RPW // _SC_CHUNK
_NBUF = 4


@functools.partial(
    pl.kernel,
    mesh=plsc.VectorSubcoreMesh(core_axis_name="c", subcore_axis_name="s"),
    out_type=jax.ShapeDtypeStruct((_ROWS, _DIM), jnp.float32),
    scratch_types=[
        pltpu.VMEM_SHARED((16, _NBUF, _SC_CHUNK, _DIM), jnp.float32),
        pltpu.SemaphoreType.DMA((_NBUF,)),
        pltpu.SemaphoreType.DMA((_NBUF,)),
    ],
)
def _sc_copy(emb_hbm, out_hbm, sbuf, in_sems, out_sems):
    sid = lax.axis_index("s")
    wid = sid * 2 + lax.axis_index("c")
    base = wid * _RPW
    buf = sbuf.at[sid]

    def in_copy(g):
        slot = g % _NBUF
        return pltpu.make_async_copy(
            emb_hbm.at[pl.ds(base + g * _SC_CHUNK, _SC_CHUNK), :],
            buf.at[slot],
            in_sems.at[slot],
        )

    def out_copy(g):
        slot = g % _NBUF
        return pltpu.make_async_copy(
            buf.at[slot],
            out_hbm.at[pl.ds(base + g * _SC_CHUNK, _SC_CHUNK), :],
            out_sems.at[slot],
        )

    in_copy(0).start()
    for g in range(_NCH):
        in_copy(g).wait()
        out_copy(g).start()
        if g + 1 < _NCH:
            if g + 1 >= _NBUF:
                out_copy(g + 1 - _NBUF).wait()
            in_copy(g + 1).start()
    for g in range(max(0, _NCH - _NBUF), _NCH):
        out_copy(g).wait()


def kernel(x, emb):
    del x
    return _sc_copy(emb)


# speedup vs baseline: 1.1739x; 1.0064x over previous
import functools

import jax
import jax.numpy as jnp
from jax import lax
from jax.experimental import pallas as pl
from jax.experimental.pallas import tpu as pltpu
from jax.experimental.pallas import tpu_sc as plsc

_ROWS = 8192
_DIM = 1024
_NW = 32
_RPW = _ROWS // _NW
_SC_CHUNK = 32
_NCH = _RPW // _SC_CHUNK
_NBUF = 4


@functools.partial(
    pl.kernel,
    mesh=plsc.VectorSubcoreMesh(core_axis_name="c", subcore_axis_name="s"),
    out_type=jax.ShapeDtypeStruct((_ROWS, _DIM), jnp.float32),
    scratch_types=[
        pltpu.VMEM_SHARED((16, _NBUF, _SC_CHUNK, _DIM), jnp.float32),
        pltpu.SemaphoreType.DMA((_NBUF,)),
        pltpu.SemaphoreType.DMA((_NBUF,)),
    ],
    compiler_params=pltpu.CompilerParams(
        skip_device_barrier=True,
        disable_bounds_checks=True,
        disable_semaphore_checks=True,
    ),
)
def _sc_copy(emb_hbm, out_hbm, sbuf, in_sems, out_sems):
    sid = lax.axis_index("s")
    wid = sid * 2 + lax.axis_index("c")
    base = wid * _RPW
    buf = sbuf.at[sid]

    def in_copy(g):
        slot = g % _NBUF
        return pltpu.make_async_copy(
            emb_hbm.at[pl.ds(base + g * _SC_CHUNK, _SC_CHUNK), :],
            buf.at[slot],
            in_sems.at[slot],
        )

    def out_copy(g):
        slot = g % _NBUF
        return pltpu.make_async_copy(
            buf.at[slot],
            out_hbm.at[pl.ds(base + g * _SC_CHUNK, _SC_CHUNK), :],
            out_sems.at[slot],
        )

    in_copy(0).start()
    for g in range(_NCH):
        in_copy(g).wait()
        out_copy(g).start()
        if g + 1 < _NCH:
            if g + 1 >= _NBUF:
                out_copy(g + 1 - _NBUF).wait()
            in_copy(g + 1).start()
    for g in range(max(0, _NCH - _NBUF), _NCH):
        out_copy(g).wait()


def kernel(x, emb):
    del x
    return _sc_copy(emb)
